# Initial kernel scaffold; baseline (speedup 1.0000x reference)
import jax
import jax.numpy as jnp
from jax.experimental import pallas as pl


def kernel(x, edge_index, edge_attr, batch, additional_features, params):
    raise NotImplementedError("write your pallas kernel here")


# speedup vs baseline: 2.9491x; 2.9491x over previous
import jax
import jax.numpy as jnp
from jax.experimental import pallas as pl
from jax.experimental.pallas import tpu as pltpu

_EPS = 1e-5


def _ln(h, g, b):
    mu = jnp.mean(h, axis=-1, keepdims=True)
    d = h - mu
    var = jnp.mean(d * d, axis=-1, keepdims=True)
    return d * jax.lax.rsqrt(var + _EPS) * g + b


def _dot(a, b):
    return jnp.dot(a, b, preferred_element_type=jnp.float32)


def kernel(x, edge_index, edge_attr, batch, additional_features, params):
    del edge_index, edge_attr
    N, node_dim = x.shape
    nseg, add_dim = additional_features.shape
    H = params['node_emb']['W'].shape[1]
    nlayers = len(params['convs'])

    R = 2048
    G = -(-N // R)
    npad = G * R
    xp = jnp.pad(x, ((0, npad - N), (0, 0)))
    bp = jnp.pad(batch, (0, npad - N), constant_values=nseg).reshape(G, 1, R)

    pe = params['node_emb']
    emb_W = pe['W']
    emb_V = jnp.stack([pe['b'], pe['g'], pe['be']])
    convs = params['convs']
    c_nW = jnp.stack([c['nW'] for c in convs])
    c_oW = jnp.stack([c['oW'] for c in convs])
    c_V = jnp.stack([jnp.stack([c['nb'], c['ng'], c['nbe'],
                                c['ob'], c['og'], c['obe']]) for c in convs])
    pa = params['add_mlp']
    a_W1, a_W2 = pa['W1'], pa['W2']
    a_V = jnp.stack([pa['b1'], pa['g'], pa['be'], pa['b2']])
    po = params['out']
    o_W1, o_W2, o_W3 = po['W1'], po['W2'], po['W3']
    o_V = jnp.stack([po['b1'], po['g'], po['be']])
    o_b2 = po['b2'].reshape(1, H)
    o_b3 = po['b3'].reshape(1, 1)

    def body(x_ref, b_ref, af_ref, embW_ref, embV_ref, nW_ref, oW_ref, cV_ref,
             aW1_ref, aW2_ref, aV_ref, oW1_ref, oW2_ref, oW3_ref, oV_ref,
             ob2_ref, ob3_ref, out_ref, acc_ref, cnt_ref):
        i = pl.program_id(0)

        @pl.when(i == 0)
        def _init():
            acc_ref[...] = jnp.zeros_like(acc_ref)
            cnt_ref[...] = jnp.zeros_like(cnt_ref)

        h = _dot(x_ref[...], embW_ref[...])
        h = jax.nn.relu(_ln(h + embV_ref[0], embV_ref[1], embV_ref[2]))
        for l in range(nlayers):
            h_t = _ln(_dot(h, nW_ref[l]) + cV_ref[l, 0], cV_ref[l, 1], cV_ref[l, 2])
            z = _dot(h, oW_ref[l, :H]) + _dot(h_t, oW_ref[l, H:]) + cV_ref[l, 3]
            h = h + _ln(z, cV_ref[l, 4], cV_ref[l, 5])

        seg = jax.lax.broadcasted_iota(jnp.int32, (nseg, R), 0)
        oh = (b_ref[0] == seg).astype(jnp.float32)
        acc_ref[...] += _dot(oh, h)
        cnt_ref[...] += jnp.sum(oh, axis=1, keepdims=True)

        @pl.when(i == pl.num_programs(0) - 1)
        def _head():
            pooled = acc_ref[...] / jnp.maximum(cnt_ref[...], 1.0)
            a = jax.nn.relu(_ln(_dot(af_ref[...], aW1_ref[...]) + aV_ref[0],
                                aV_ref[1], aV_ref[2]))
            a = _dot(a, aW2_ref[...]) + aV_ref[3]
            z = _dot(pooled, oW1_ref[:H]) + _dot(a, oW1_ref[H:]) + oV_ref[0]
            o = jax.nn.relu(_ln(z, oV_ref[1], oV_ref[2]))
            o = jax.nn.relu(_dot(o, oW2_ref[...]) + ob2_ref[...])
            out_ref[...] = _dot(o, oW3_ref[...]) + ob3_ref[...]

    def const_spec(a):
        nd = a.ndim
        return pl.BlockSpec(a.shape, lambda i, _n=nd: (0,) * _n)

    weights = [emb_W, emb_V, c_nW, c_oW, c_V, a_W1, a_W2, a_V,
               o_W1, o_W2, o_W3, o_V, o_b2, o_b3]
    in_specs = [
        pl.BlockSpec((R, node_dim), lambda i: (i, 0)),
        pl.BlockSpec((1, 1, R), lambda i: (i, 0, 0)),
        const_spec(additional_features),
    ] + [const_spec(w) for w in weights]

    return pl.pallas_call(
        body,
        grid=(G,),
        in_specs=in_specs,
        out_specs=pl.BlockSpec((nseg, 1), lambda i: (0, 0)),
        out_shape=jax.ShapeDtypeStruct((nseg, 1), jnp.float32),
        scratch_shapes=[pltpu.VMEM((nseg, H), jnp.float32),
                        pltpu.VMEM((nseg, 1), jnp.float32)],
    )(xp, bp, additional_features, *weights)
